# Initial kernel scaffold; baseline (speedup 1.0000x reference)
import jax
import jax.numpy as jnp
from jax.experimental import pallas as pl


def kernel(inputs, weights):
    raise NotImplementedError("write your pallas kernel here")


# speedup vs baseline: 1.6315x; 1.6315x over previous
import jax
import jax.numpy as jnp
from jax.experimental import pallas as pl
from jax.experimental.pallas import tpu as pltpu

_COMMIT = 0.25
_BN = 128


def _vq_kernel(x_ref, w_ref,
               dist_ref, oh_ref, emb_ref, qst_ref, loss_ref, perp_ref,
               counts_ref, ws_ref, acc_ref):
    g = pl.program_id(0)
    i = pl.program_id(1)
    ng = pl.num_programs(0)
    nb = pl.num_programs(1)

    x = x_ref[0]
    w = w_ref[0]
    bn, d = x.shape
    k = w.shape[0]

    @pl.when(jnp.logical_and(g == 0, i == 0))
    def _():
        acc_ref[0] = 0.0
        acc_ref[1] = 0.0

    @pl.when(i == 0)
    def _():
        counts_ref[...] = jnp.zeros_like(counts_ref)
        w2 = w * w
        ones = jnp.ones((1, d), jnp.float32)
        ws_ref[...] = jax.lax.dot_general(
            ones, w2, (((1,), (1,)), ((), ())),
            preferred_element_type=jnp.float32)

    xs = jnp.sum(x * x, axis=1, keepdims=True)
    dots = jax.lax.dot_general(
        x, w, (((1,), (1,)), ((), ())),
        preferred_element_type=jnp.float32)
    dist = xs + ws_ref[...] - 2.0 * dots
    dist_ref[0] = dist

    mind = jnp.min(dist, axis=1, keepdims=True)
    iota = jax.lax.broadcasted_iota(jnp.int32, (bn, k), 1)
    idx = jnp.min(jnp.where(dist == mind, iota, k),
                  axis=1, keepdims=True)
    oh = (iota == idx).astype(jnp.float32)
    oh_ref[0] = oh
    counts_ref[...] += jnp.sum(oh, axis=0, keepdims=True)

    quant = jax.lax.dot_general(
        oh, w, (((1,), (0,)), ((), ())),
        preferred_element_type=jnp.float32)
    emb_ref[0] = quant
    qst_ref[0] = x + (quant - x)

    diff = quant - x
    acc_ref[0] += jnp.sum(diff * diff)

    @pl.when(i == nb - 1)
    def _():
        n_tok = nb * bn
        p = counts_ref[...] * (1.0 / n_tok)
        ent = -jnp.sum(p * jnp.log(p + 1e-10))
        acc_ref[1] += jnp.exp(ent)

    @pl.when(jnp.logical_and(g == ng - 1, i == nb - 1))
    def _():
        n_tok = nb * bn
        loss_ref[...] = jnp.full(
            (1, 1), acc_ref[0] * ((1.0 + _COMMIT) / (ng * n_tok * d)),
            dtype=jnp.float32)
        perp_ref[...] = jnp.full((1, 1), acc_ref[1] * (1.0 / ng),
                                 dtype=jnp.float32)


def kernel(inputs, weights):
    n, g, d = inputs.shape
    _, k, _ = weights.shape
    bn = _BN
    x = jnp.transpose(inputs, (1, 0, 2))

    grid = (g, n // bn)
    out_shape = (
        jax.ShapeDtypeStruct((g, n, k), jnp.float32),
        jax.ShapeDtypeStruct((g, n, k), jnp.float32),
        jax.ShapeDtypeStruct((g, n, d), jnp.float32),
        jax.ShapeDtypeStruct((g, n, d), jnp.float32),
        jax.ShapeDtypeStruct((1, 1), jnp.float32),
        jax.ShapeDtypeStruct((1, 1), jnp.float32),
    )
    in_specs = [
        pl.BlockSpec((1, bn, d), lambda gi, ii: (gi, ii, 0)),
        pl.BlockSpec((1, k, d), lambda gi, ii: (gi, 0, 0)),
    ]
    out_specs = (
        pl.BlockSpec((1, bn, k), lambda gi, ii: (gi, ii, 0)),
        pl.BlockSpec((1, bn, k), lambda gi, ii: (gi, ii, 0)),
        pl.BlockSpec((1, bn, d), lambda gi, ii: (gi, ii, 0)),
        pl.BlockSpec((1, bn, d), lambda gi, ii: (gi, ii, 0)),
        pl.BlockSpec((1, 1), lambda gi, ii: (0, 0)),
        pl.BlockSpec((1, 1), lambda gi, ii: (0, 0)),
    )
    scratch_shapes = [
        pltpu.VMEM((1, k), jnp.float32),
        pltpu.VMEM((1, k), jnp.float32),
        pltpu.SMEM((2,), jnp.float32),
    ]
    dist, oh, emb, qst, loss, perp = pl.pallas_call(
        _vq_kernel,
        grid=grid,
        in_specs=in_specs,
        out_specs=out_specs,
        out_shape=out_shape,
        scratch_shapes=scratch_shapes,
    )(x, weights)
    quantized_all = jnp.transpose(qst, (1, 0, 2))
    return (loss[0, 0], quantized_all, perp[0, 0], emb, oh, dist)
